# Initial kernel scaffold; baseline (speedup 1.0000x reference)
import jax
import jax.numpy as jnp
from jax.experimental import pallas as pl


def kernel(inputs, token_table, pos_table):
    raise NotImplementedError("write your pallas kernel here")


# speedup vs baseline: 1.3894x; 1.3894x over previous
import functools

import jax
import jax.numpy as jnp
from jax import lax
from jax.experimental import pallas as pl
from jax.experimental.pallas import tpu as pltpu
from jax.experimental.pallas import tpu_sc as plsc

NC = 2
NS = 16
NW = NC * NS
LANES = 16


@functools.lru_cache(maxsize=None)
def _build(B, L, V, D):
    BL = B * L
    SEQ_PER_W = B // NW
    CS = 4
    CHUNK_ROWS = CS * L
    NCHUNK = SEQ_PER_W // CS
    NSTREAM = 8
    SROWS = CHUNK_ROWS // NSTREAM
    DV = D // LANES

    assert B % NW == 0 and SEQ_PER_W % CS == 0
    assert CHUNK_ROWS % NSTREAM == 0 and SROWS <= 128
    assert D % LANES == 0

    mesh = plsc.VectorSubcoreMesh(core_axis_name="c", subcore_axis_name="s")

    @functools.partial(
        pl.kernel,
        out_type=jax.ShapeDtypeStruct((BL, D), jnp.float32),
        mesh=mesh,
        compiler_params=pltpu.CompilerParams(use_tc_tiling_on_sc=False),
        scratch_types=[
            pltpu.VMEM((NSTREAM, SROWS), jnp.int32),
            pltpu.VMEM((CHUNK_ROWS, D), jnp.float32),
            pltpu.VMEM((L, D), jnp.float32),
            pltpu.SemaphoreType.DMA,
        ],
    )
    def gather_add(idx_hbm, table_hbm, pos_hbm, out_hbm, idx_v, rows_v, pos_v, sem):
        wid = lax.axis_index("s") * NC + lax.axis_index("c")
        pltpu.sync_copy(pos_hbm, pos_v)
        base_row = wid * (SEQ_PER_W * L)

        def chunk_body(c, carry):
            row0 = pl.multiple_of(base_row + c * CHUNK_ROWS, 8)
            r0s = pl.multiple_of((base_row + c * CHUNK_ROWS) // SROWS, 8)
            pltpu.sync_copy(idx_hbm.at[pl.ds(r0s, NSTREAM)], idx_v)
            copies = [
                pltpu.async_copy(
                    table_hbm.at[idx_v.at[j]],
                    rows_v.at[pl.ds(j * SROWS, SROWS)],
                    sem,
                )
                for j in range(NSTREAM)
            ]
            for cp in copies:
                cp.wait()

            def add_l(l, _):
                for d in range(DV):
                    pv = pos_v[l, pl.ds(d * LANES, LANES)]
                    for s in range(CS):
                        rows_v[s * L + l, pl.ds(d * LANES, LANES)] += pv
                return _

            lax.fori_loop(0, L, add_l, 0)
            pltpu.sync_copy(rows_v, out_hbm.at[pl.ds(row0, CHUNK_ROWS)])
            return carry

        lax.fori_loop(0, NCHUNK, chunk_body, 0)

    return gather_add


def kernel(inputs, token_table, pos_table):
    B, L = inputs.shape
    V, D = token_table.shape
    BL = B * L
    SROWS = (4 * L) // 8
    idx2d = inputs.astype(jnp.int32).reshape(BL // SROWS, SROWS)
    out = _build(B, L, V, D)(idx2d, token_table, pos_table)
    return out.reshape(B, L, D)
